# baseline (speedup 1.0000x reference)
import functools

import jax
import jax.numpy as jnp
from jax import lax
from jax.experimental import pallas as pl
from jax.experimental.pallas import tpu as pltpu
from jax.experimental.pallas import tpu_sc as plsc

E = 8
H = 1024
HW = H // 2
FF = 512
T = 2048
BK = 128
NB = (2 * T) // BK + E - 1
P = NB * BK
LANES = 16
WREP = 128

NC = 2
NS = 16
NW = NC * NS
TPW = T // NW
CH = 16


def _router_kernel(x_ref, gate_ref, logits_ref, p0_ref, p1_ref,
                   w0s_ref, w1s_ref, blk_ref):
    x = x_ref[...]
    logits = lax.dot_general(x, gate_ref[...], (((1,), (1,)), ((), ())),
                             preferred_element_type=jnp.float32)
    logits_ref[...] = logits
    scores = jax.nn.sigmoid(logits)
    i1 = jnp.argmax(scores, axis=1)
    v1 = jnp.max(scores, axis=1)
    cols = lax.broadcasted_iota(jnp.int32, scores.shape, 1)
    masked = jnp.where(cols == i1[:, None], -jnp.inf, scores)
    i2 = jnp.argmax(masked, axis=1)
    v2 = jnp.max(masked, axis=1)
    denom = v1 + v2 + 1e-6
    wa = (v1 / denom)[:, None]
    wb = (v2 / denom)[:, None]
    oh0 = (cols == i1[:, None]).astype(jnp.float32)
    oh1 = (cols == i2[:, None]).astype(jnp.float32)
    cnt = oh0 + oh1

    nch = T // BK
    r = lax.broadcasted_iota(jnp.int32, (BK, BK), 0)
    c = lax.broadcasted_iota(jnp.int32, (BK, BK), 1)
    ltri = (c < r).astype(jnp.float32)
    run = jnp.zeros((1, E), jnp.float32)
    cume_parts = []
    for i in range(nch):
        ci = lax.slice(cnt, (i * BK, 0), ((i + 1) * BK, E))
        part = lax.dot_general(ltri, ci, (((1,), (0,)), ((), ())),
                               preferred_element_type=jnp.float32)
        cume_parts.append(part + run)
        run = run + jnp.sum(ci, axis=0, keepdims=True)
    cume = jnp.concatenate(cume_parts, axis=0)

    totals = run
    nblk = jnp.floor((totals + (BK - 1)) * (1.0 / BK))
    e1 = lax.broadcasted_iota(jnp.int32, (E, E), 0)
    e2 = lax.broadcasted_iota(jnp.int32, (E, E), 1)
    utri = (e1 < e2).astype(jnp.float32)
    bstart = lax.dot_general(nblk, utri, (((1,), (0,)), ((), ())),
                             preferred_element_type=jnp.float32)
    poff = bstart * BK

    bs_col = jnp.transpose(bstart).astype(jnp.int32)
    jj = lax.broadcasted_iota(jnp.int32, (E, NB), 1)
    blk = jnp.sum((jj >= bs_col).astype(jnp.int32), axis=0,
                  keepdims=True) - 1
    blk_ref[...] = blk

    pos_all = cume + poff
    p0 = jnp.sum(oh0 * pos_all, axis=1, keepdims=True)
    p1 = jnp.sum(oh1 * pos_all, axis=1, keepdims=True)
    p0_ref[...] = p0.astype(jnp.int32)
    p1_ref[...] = p1.astype(jnp.int32)

    ones = jnp.ones((1, WREP), jnp.float32)
    w0s_ref[...] = wa * ones
    w1s_ref[...] = wb * ones


def _run_router(x, gate_w):
    return pl.pallas_call(
        _router_kernel,
        out_shape=[
            jax.ShapeDtypeStruct((T, E), jnp.float32),
            jax.ShapeDtypeStruct((T, 1), jnp.int32),
            jax.ShapeDtypeStruct((T, 1), jnp.int32),
            jax.ShapeDtypeStruct((T, WREP), jnp.float32),
            jax.ShapeDtypeStruct((T, WREP), jnp.float32),
            jax.ShapeDtypeStruct((1, NB), jnp.int32),
        ],
    )(x, gate_w)


def _scatter_kernel(xb_hbm, p0_hbm, p1_hbm, w0s_hbm, w1s_hbm,
                    xg_hbm, wg_hbm, xr, i0, i1, wr0, wr1, sem):
    wid = lax.axis_index("s") * NC + lax.axis_index("c")
    base = wid * TPW
    pltpu.sync_copy(xb_hbm.at[pl.ds(base, TPW)], xr)
    pltpu.sync_copy(p0_hbm.at[pl.ds(base, TPW)], i0)
    pltpu.sync_copy(p1_hbm.at[pl.ds(base, TPW)], i1)
    pltpu.sync_copy(w0s_hbm.at[pl.ds(base, TPW)], wr0)
    pltpu.sync_copy(w1s_hbm.at[pl.ds(base, TPW)], wr1)
    c0 = pltpu.async_copy(xr, xg_hbm.at[i0], sem)
    c1 = pltpu.async_copy(xr, xg_hbm.at[i1], sem)
    c2 = pltpu.async_copy(wr0, wg_hbm.at[i0], sem)
    c3 = pltpu.async_copy(wr1, wg_hbm.at[i1], sem)
    c0.wait()
    c1.wait()
    c2.wait()
    c3.wait()


def _run_scatter(xbi, p0, p1, w0s, w1s):
    mesh = plsc.VectorSubcoreMesh(core_axis_name="c", subcore_axis_name="s")
    f = pl.kernel(
        _scatter_kernel,
        mesh=mesh,
        out_type=[
            jax.ShapeDtypeStruct((P, H), jnp.float32),
            jax.ShapeDtypeStruct((P, WREP), jnp.float32),
        ],
        scratch_types=[
            pltpu.VMEM((TPW, H), jnp.float32),
            pltpu.VMEM((TPW,), jnp.int32),
            pltpu.VMEM((TPW,), jnp.int32),
            pltpu.VMEM((TPW, WREP), jnp.float32),
            pltpu.VMEM((TPW, WREP), jnp.float32),
            pltpu.SemaphoreType.DMA,
        ],
    )
    return f(xbi, p0, p1, w0s, w1s)


def _ffn_kernel(blk_ref, xg_ref, wg_ref, w1_ref, w3_ref, w2_ref, yg_ref):
    xb = xg_ref[...]
    h1 = lax.dot_general(xb, w1_ref[0], (((1,), (1,)), ((), ())),
                         preferred_element_type=jnp.float32)
    h3 = lax.dot_general(xb, w3_ref[0], (((1,), (1,)), ((), ())),
                         preferred_element_type=jnp.float32)
    he = (h1 * jax.nn.sigmoid(h1)) * h3
    ye = lax.dot_general(he, w2_ref[0], (((1,), (1,)), ((), ())),
                         preferred_element_type=jnp.float32)
    yg_ref[...] = ye * wg_ref[:, :1]


def _run_ffn(blk, xgb, wg, w1, w3, w2):
    grid_spec = pltpu.PrefetchScalarGridSpec(
        num_scalar_prefetch=1,
        grid=(NB,),
        in_specs=[
            pl.BlockSpec((BK, H), lambda j, blk: (j, 0)),
            pl.BlockSpec((BK, WREP), lambda j, blk: (j, 0)),
            pl.BlockSpec((1, FF, H), lambda j, blk: (blk[j], 0, 0)),
            pl.BlockSpec((1, FF, H), lambda j, blk: (blk[j], 0, 0)),
            pl.BlockSpec((1, H, FF), lambda j, blk: (blk[j], 0, 0)),
        ],
        out_specs=pl.BlockSpec((BK, H), lambda j, blk: (j, 0)),
    )
    return pl.pallas_call(
        _ffn_kernel,
        grid_spec=grid_spec,
        out_shape=jax.ShapeDtypeStruct((P, H), jnp.float32),
    )(blk, xgb, wg, w1, w3, w2)


def _combine_kernel(yg_hbm, p0_hbm, p1_hbm, out_hbm,
                    ia, ib, b0a, b0b, b1a, b1b, ob0, ob1, sem, osem):
    wid = lax.axis_index("s") * NC + lax.axis_index("c")
    base = wid * TPW
    nch = TPW // CH
    pltpu.sync_copy(p0_hbm.at[pl.ds(base, TPW)], ia)
    pltpu.sync_copy(p1_hbm.at[pl.ds(base, TPW)], ib)
    bufs = ((b0a, b0b, ob0), (b1a, b1b, ob1))
    pend = {}
    pend[0] = (pltpu.async_copy(yg_hbm.at[ia.at[pl.ds(0, CH)]], b0a, sem),
               pltpu.async_copy(yg_hbm.at[ib.at[pl.ds(0, CH)]], b0b, sem))
    st = []
    for cidx in range(nch):
        ba, bb, ob = bufs[cidx % 2]
        if cidx + 1 < nch:
            na, nb, _ = bufs[(cidx + 1) % 2]
            s = (cidx + 1) * CH
            pend[cidx + 1] = (
                pltpu.async_copy(yg_hbm.at[ia.at[pl.ds(s, CH)]], na, sem),
                pltpu.async_copy(yg_hbm.at[ib.at[pl.ds(s, CH)]], nb, sem))
        ga, gb = pend.pop(cidx)
        ga.wait()
        gb.wait()
        if len(st) >= 2:
            st.pop(0).wait()

        def body(k, _):
            s = k * LANES
            for r in range(CH):
                ob[r, pl.ds(s, LANES)] = (ba[r, pl.ds(s, LANES)]
                                          + bb[r, pl.ds(s, LANES)])
            return 0

        lax.fori_loop(0, H // LANES, body, 0)
        st.append(pltpu.async_copy(
            ob, out_hbm.at[pl.ds(base + cidx * CH, CH)], osem))
    for h in st:
        h.wait()


def _run_combine(ygi, p0, p1):
    mesh = plsc.VectorSubcoreMesh(core_axis_name="c", subcore_axis_name="s")
    f = pl.kernel(
        _combine_kernel,
        mesh=mesh,
        out_type=jax.ShapeDtypeStruct((T, H), jnp.float32),
        scratch_types=[
            pltpu.VMEM((TPW,), jnp.int32),
            pltpu.VMEM((TPW,), jnp.int32),
            pltpu.VMEM((CH, H), jnp.float32),
            pltpu.VMEM((CH, H), jnp.float32),
            pltpu.VMEM((CH, H), jnp.float32),
            pltpu.VMEM((CH, H), jnp.float32),
            pltpu.VMEM((CH, H), jnp.float32),
            pltpu.VMEM((CH, H), jnp.float32),
            pltpu.SemaphoreType.DMA,
            pltpu.SemaphoreType.DMA,
        ],
    )
    return f(ygi, p0, p1)


def kernel(hidden_states, gate_w, w1, w3, w2):
    b, s, h = hidden_states.shape
    x = hidden_states.reshape(-1, h)
    logits, p0, p1, w0s, w1s, blk = _run_router(x, gate_w)
    p0 = p0.reshape(T)
    p1 = p1.reshape(T)
    blk = blk.reshape(NB)
    xg, wg = _run_scatter(x, p0, p1, w0s, w1s)
    yg = _run_ffn(blk, xg, wg, w1, w3, w2)
    out = _run_combine(yg, p0, p1)
    return out.reshape(b, s, h), logits


# speedup vs baseline: 1.1823x; 1.1823x over previous
import functools

import jax
import jax.numpy as jnp
from jax import lax
from jax.experimental import pallas as pl
from jax.experimental.pallas import tpu as pltpu
from jax.experimental.pallas import tpu_sc as plsc

E = 8
H = 1024
HW = H // 2
FF = 512
T = 2048
BK = 256
NB = (2 * T) // BK + E - 1
P = NB * BK
LANES = 16
WREP = 128

NC = 2
NS = 16
NW = NC * NS
TPW = T // NW
CH = 16


def _router_kernel(x_ref, gate_ref, logits_ref, p0_ref, p1_ref,
                   w0s_ref, w1s_ref, blk_ref):
    x = x_ref[...]
    logits = lax.dot_general(x, gate_ref[...], (((1,), (1,)), ((), ())),
                             preferred_element_type=jnp.float32)
    logits_ref[...] = logits
    scores = jax.nn.sigmoid(logits)
    i1 = jnp.argmax(scores, axis=1)
    v1 = jnp.max(scores, axis=1)
    cols = lax.broadcasted_iota(jnp.int32, scores.shape, 1)
    masked = jnp.where(cols == i1[:, None], -jnp.inf, scores)
    i2 = jnp.argmax(masked, axis=1)
    v2 = jnp.max(masked, axis=1)
    denom = v1 + v2 + 1e-6
    wa = (v1 / denom)[:, None]
    wb = (v2 / denom)[:, None]
    oh0 = (cols == i1[:, None]).astype(jnp.float32)
    oh1 = (cols == i2[:, None]).astype(jnp.float32)
    cnt = oh0 + oh1

    nch = T // BK
    r = lax.broadcasted_iota(jnp.int32, (BK, BK), 0)
    c = lax.broadcasted_iota(jnp.int32, (BK, BK), 1)
    ltri = (c < r).astype(jnp.float32)
    run = jnp.zeros((1, E), jnp.float32)
    cume_parts = []
    for i in range(nch):
        ci = lax.slice(cnt, (i * BK, 0), ((i + 1) * BK, E))
        part = lax.dot_general(ltri, ci, (((1,), (0,)), ((), ())),
                               preferred_element_type=jnp.float32)
        cume_parts.append(part + run)
        run = run + jnp.sum(ci, axis=0, keepdims=True)
    cume = jnp.concatenate(cume_parts, axis=0)

    totals = run
    nblk = jnp.floor((totals + (BK - 1)) * (1.0 / BK))
    e1 = lax.broadcasted_iota(jnp.int32, (E, E), 0)
    e2 = lax.broadcasted_iota(jnp.int32, (E, E), 1)
    utri = (e1 < e2).astype(jnp.float32)
    bstart = lax.dot_general(nblk, utri, (((1,), (0,)), ((), ())),
                             preferred_element_type=jnp.float32)
    poff = bstart * BK

    bs_col = jnp.transpose(bstart).astype(jnp.int32)
    jj = lax.broadcasted_iota(jnp.int32, (E, NB), 1)
    blk = jnp.sum((jj >= bs_col).astype(jnp.int32), axis=0,
                  keepdims=True) - 1
    blk_ref[...] = blk

    pos_all = cume + poff
    p0 = jnp.sum(oh0 * pos_all, axis=1, keepdims=True)
    p1 = jnp.sum(oh1 * pos_all, axis=1, keepdims=True)
    p0_ref[...] = p0.astype(jnp.int32)
    p1_ref[...] = p1.astype(jnp.int32)

    ones = jnp.ones((1, WREP), jnp.float32)
    w0s_ref[...] = wa * ones
    w1s_ref[...] = wb * ones


def _run_router(x, gate_w):
    return pl.pallas_call(
        _router_kernel,
        out_shape=[
            jax.ShapeDtypeStruct((T, E), jnp.float32),
            jax.ShapeDtypeStruct((T, 1), jnp.int32),
            jax.ShapeDtypeStruct((T, 1), jnp.int32),
            jax.ShapeDtypeStruct((T, WREP), jnp.float32),
            jax.ShapeDtypeStruct((T, WREP), jnp.float32),
            jax.ShapeDtypeStruct((1, NB), jnp.int32),
        ],
    )(x, gate_w)


def _scatter_kernel(xb_hbm, p0_hbm, p1_hbm, w0s_hbm, w1s_hbm,
                    xg_hbm, wg_hbm, xr0, xr1, i0a, i0b, i1a, i1b,
                    wr0a, wr0b, wr1a, wr1b, lsem, sem):
    wid = lax.axis_index("s") * NC + lax.axis_index("c")
    base = wid * TPW
    hpw = TPW // 2
    l0 = pltpu.async_copy(xb_hbm.at[pl.ds(base, hpw)], xr0, lsem)
    l1 = pltpu.async_copy(xb_hbm.at[pl.ds(base + hpw, hpw)], xr1, lsem)
    pltpu.sync_copy(p0_hbm.at[pl.ds(base, hpw)], i0a)
    pltpu.sync_copy(p0_hbm.at[pl.ds(base + hpw, hpw)], i0b)
    pltpu.sync_copy(p1_hbm.at[pl.ds(base, hpw)], i1a)
    pltpu.sync_copy(p1_hbm.at[pl.ds(base + hpw, hpw)], i1b)
    pltpu.sync_copy(w0s_hbm.at[pl.ds(base, hpw)], wr0a)
    pltpu.sync_copy(w0s_hbm.at[pl.ds(base + hpw, hpw)], wr0b)
    pltpu.sync_copy(w1s_hbm.at[pl.ds(base, hpw)], wr1a)
    pltpu.sync_copy(w1s_hbm.at[pl.ds(base + hpw, hpw)], wr1b)
    l0.wait()
    c0 = pltpu.async_copy(xr0, xg_hbm.at[i0a], sem)
    c1 = pltpu.async_copy(xr0, xg_hbm.at[i1a], sem)
    l1.wait()
    c2 = pltpu.async_copy(xr1, xg_hbm.at[i0b], sem)
    c3 = pltpu.async_copy(xr1, xg_hbm.at[i1b], sem)
    c4 = pltpu.async_copy(wr0a, wg_hbm.at[i0a], sem)
    c5 = pltpu.async_copy(wr1a, wg_hbm.at[i1a], sem)
    c6 = pltpu.async_copy(wr0b, wg_hbm.at[i0b], sem)
    c7 = pltpu.async_copy(wr1b, wg_hbm.at[i1b], sem)
    for c in (c0, c1, c2, c3, c4, c5, c6, c7):
        c.wait()


def _run_scatter(xbi, p0, p1, w0s, w1s):
    mesh = plsc.VectorSubcoreMesh(core_axis_name="c", subcore_axis_name="s")
    f = pl.kernel(
        _scatter_kernel,
        mesh=mesh,
        out_type=[
            jax.ShapeDtypeStruct((P, H), jnp.float32),
            jax.ShapeDtypeStruct((P, WREP), jnp.float32),
        ],
        scratch_types=[
            pltpu.VMEM((TPW // 2, H), jnp.float32),
            pltpu.VMEM((TPW // 2, H), jnp.float32),
            pltpu.VMEM((TPW // 2,), jnp.int32),
            pltpu.VMEM((TPW // 2,), jnp.int32),
            pltpu.VMEM((TPW // 2,), jnp.int32),
            pltpu.VMEM((TPW // 2,), jnp.int32),
            pltpu.VMEM((TPW // 2, WREP), jnp.float32),
            pltpu.VMEM((TPW // 2, WREP), jnp.float32),
            pltpu.VMEM((TPW // 2, WREP), jnp.float32),
            pltpu.VMEM((TPW // 2, WREP), jnp.float32),
            pltpu.SemaphoreType.DMA,
            pltpu.SemaphoreType.DMA,
        ],
    )
    return f(xbi, p0, p1, w0s, w1s)


def _ffn_kernel(blk_ref, xg_ref, wg_ref, w1_ref, w3_ref, w2_ref, yg_ref):
    xb = xg_ref[...]
    h1 = lax.dot_general(xb, w1_ref[0], (((1,), (1,)), ((), ())),
                         preferred_element_type=jnp.float32)
    h3 = lax.dot_general(xb, w3_ref[0], (((1,), (1,)), ((), ())),
                         preferred_element_type=jnp.float32)
    he = (h1 * jax.nn.sigmoid(h1)) * h3
    ye = lax.dot_general(he, w2_ref[0], (((1,), (1,)), ((), ())),
                         preferred_element_type=jnp.float32)
    yg_ref[...] = ye * wg_ref[:, :1]


def _run_ffn(blk, xgb, wg, w1, w3, w2):
    grid_spec = pltpu.PrefetchScalarGridSpec(
        num_scalar_prefetch=1,
        grid=(NB,),
        in_specs=[
            pl.BlockSpec((BK, H), lambda j, blk: (j, 0)),
            pl.BlockSpec((BK, WREP), lambda j, blk: (j, 0)),
            pl.BlockSpec((1, FF, H), lambda j, blk: (blk[j], 0, 0)),
            pl.BlockSpec((1, FF, H), lambda j, blk: (blk[j], 0, 0)),
            pl.BlockSpec((1, H, FF), lambda j, blk: (blk[j], 0, 0)),
        ],
        out_specs=pl.BlockSpec((BK, H), lambda j, blk: (j, 0)),
    )
    return pl.pallas_call(
        _ffn_kernel,
        grid_spec=grid_spec,
        out_shape=jax.ShapeDtypeStruct((P, H), jnp.float32),
    )(blk, xgb, wg, w1, w3, w2)


def _combine_kernel(yg_hbm, p0_hbm, p1_hbm, out_hbm,
                    ia, ib, b0a, b0b, b1a, b1b, ob0, ob1, sem, osem):
    wid = lax.axis_index("s") * NC + lax.axis_index("c")
    base = wid * TPW
    nch = TPW // CH
    pltpu.sync_copy(p0_hbm.at[pl.ds(base, TPW)], ia)
    pltpu.sync_copy(p1_hbm.at[pl.ds(base, TPW)], ib)
    bufs = ((b0a, b0b, ob0), (b1a, b1b, ob1))
    pend = {}
    pend[0] = (pltpu.async_copy(yg_hbm.at[ia.at[pl.ds(0, CH)]], b0a, sem),
               pltpu.async_copy(yg_hbm.at[ib.at[pl.ds(0, CH)]], b0b, sem))
    st = []
    for cidx in range(nch):
        ba, bb, ob = bufs[cidx % 2]
        if cidx + 1 < nch:
            na, nb, _ = bufs[(cidx + 1) % 2]
            s = (cidx + 1) * CH
            pend[cidx + 1] = (
                pltpu.async_copy(yg_hbm.at[ia.at[pl.ds(s, CH)]], na, sem),
                pltpu.async_copy(yg_hbm.at[ib.at[pl.ds(s, CH)]], nb, sem))
        ga, gb = pend.pop(cidx)
        ga.wait()
        gb.wait()
        if len(st) >= 2:
            st.pop(0).wait()

        def body(k, _):
            s = k * LANES
            for r in range(CH):
                ob[r, pl.ds(s, LANES)] = (ba[r, pl.ds(s, LANES)]
                                          + bb[r, pl.ds(s, LANES)])
            return 0

        lax.fori_loop(0, H // LANES, body, 0)
        st.append(pltpu.async_copy(
            ob, out_hbm.at[pl.ds(base + cidx * CH, CH)], osem))
    for h in st:
        h.wait()


def _run_combine(ygi, p0, p1):
    mesh = plsc.VectorSubcoreMesh(core_axis_name="c", subcore_axis_name="s")
    f = pl.kernel(
        _combine_kernel,
        mesh=mesh,
        out_type=jax.ShapeDtypeStruct((T, H), jnp.float32),
        scratch_types=[
            pltpu.VMEM((TPW,), jnp.int32),
            pltpu.VMEM((TPW,), jnp.int32),
            pltpu.VMEM((CH, H), jnp.float32),
            pltpu.VMEM((CH, H), jnp.float32),
            pltpu.VMEM((CH, H), jnp.float32),
            pltpu.VMEM((CH, H), jnp.float32),
            pltpu.VMEM((CH, H), jnp.float32),
            pltpu.VMEM((CH, H), jnp.float32),
            pltpu.SemaphoreType.DMA,
            pltpu.SemaphoreType.DMA,
        ],
    )
    return f(ygi, p0, p1)


def kernel(hidden_states, gate_w, w1, w3, w2):
    b, s, h = hidden_states.shape
    x = hidden_states.reshape(-1, h)
    logits, p0, p1, w0s, w1s, blk = _run_router(x, gate_w)
    p0 = p0.reshape(T)
    p1 = p1.reshape(T)
    blk = blk.reshape(NB)
    xg, wg = _run_scatter(x, p0, p1, w0s, w1s)
    yg = _run_ffn(blk, xg, wg, w1, w3, w2)
    out = _run_combine(yg, p0, p1)
    return out.reshape(b, s, h), logits


# speedup vs baseline: 1.1855x; 1.0027x over previous
import jax
import jax.numpy as jnp
from jax import lax
from jax.experimental import pallas as pl
from jax.experimental.pallas import tpu as pltpu
from jax.experimental.pallas import tpu_sc as plsc

E = 8
H = 1024
FF = 512
T = 2048
BK = 256
NB = (2 * T) // BK + E - 1
P = NB * BK
LANES = 16
WREP = 128

NC = 2
NS = 16
NW = NC * NS
TPW = T // NW
CH = 16


def _router_kernel(x_ref, gate_ref, logits_ref, p0_ref, p1_ref,
                   w0s_ref, w1s_ref, blk_ref):
    x = x_ref[...]
    logits = lax.dot_general(x, gate_ref[...], (((1,), (1,)), ((), ())),
                             preferred_element_type=jnp.float32)
    logits_ref[...] = logits
    scores = jax.nn.sigmoid(logits)
    i1 = jnp.argmax(scores, axis=1)
    v1 = jnp.max(scores, axis=1)
    cols = lax.broadcasted_iota(jnp.int32, scores.shape, 1)
    masked = jnp.where(cols == i1[:, None], -jnp.inf, scores)
    i2 = jnp.argmax(masked, axis=1)
    v2 = jnp.max(masked, axis=1)
    denom = v1 + v2 + 1e-6
    wa = (v1 / denom)[:, None]
    wb = (v2 / denom)[:, None]
    oh0 = (cols == i1[:, None]).astype(jnp.float32)
    oh1 = (cols == i2[:, None]).astype(jnp.float32)
    cnt = oh0 + oh1

    nch = T // BK
    r = lax.broadcasted_iota(jnp.int32, (BK, BK), 0)
    c = lax.broadcasted_iota(jnp.int32, (BK, BK), 1)
    ltri = (c < r).astype(jnp.float32)
    run = jnp.zeros((1, E), jnp.float32)
    cume_parts = []
    for i in range(nch):
        ci = lax.slice(cnt, (i * BK, 0), ((i + 1) * BK, E))
        part = lax.dot_general(ltri, ci, (((1,), (0,)), ((), ())),
                               preferred_element_type=jnp.float32)
        cume_parts.append(part + run)
        run = run + jnp.sum(ci, axis=0, keepdims=True)
    cume = jnp.concatenate(cume_parts, axis=0)

    totals = run
    nblk = jnp.floor((totals + (BK - 1)) * (1.0 / BK))
    e1 = lax.broadcasted_iota(jnp.int32, (E, E), 0)
    e2 = lax.broadcasted_iota(jnp.int32, (E, E), 1)
    utri = (e1 < e2).astype(jnp.float32)
    bstart = lax.dot_general(nblk, utri, (((1,), (0,)), ((), ())),
                             preferred_element_type=jnp.float32)
    poff = bstart * BK

    bs_col = jnp.transpose(bstart).astype(jnp.int32)
    jj = lax.broadcasted_iota(jnp.int32, (E, NB), 1)
    blk = jnp.sum((jj >= bs_col).astype(jnp.int32), axis=0,
                  keepdims=True) - 1
    blk_ref[...] = blk

    pos_all = cume + poff
    p0 = jnp.sum(oh0 * pos_all, axis=1, keepdims=True)
    p1 = jnp.sum(oh1 * pos_all, axis=1, keepdims=True)
    p0_ref[...] = p0.astype(jnp.int32)
    p1_ref[...] = p1.astype(jnp.int32)

    ones = jnp.ones((1, WREP), jnp.float32)
    w0s_ref[...] = wa * ones
    w1s_ref[...] = wb * ones


def _run_router(x, gate_w):
    return pl.pallas_call(
        _router_kernel,
        out_shape=[
            jax.ShapeDtypeStruct((T, E), jnp.float32),
            jax.ShapeDtypeStruct((T, 1), jnp.int32),
            jax.ShapeDtypeStruct((T, 1), jnp.int32),
            jax.ShapeDtypeStruct((T, WREP), jnp.float32),
            jax.ShapeDtypeStruct((T, WREP), jnp.float32),
            jax.ShapeDtypeStruct((1, NB), jnp.int32),
        ],
    )(x, gate_w)


def _scatter_kernel(xb_hbm, p0_hbm, p1_hbm, w0s_hbm, w1s_hbm,
                    xg_hbm, wg_hbm, xr0, xr1, i0a, i0b, i1a, i1b,
                    wr0a, wr0b, wr1a, wr1b, lsem, sem):
    wid = lax.axis_index("s") * NC + lax.axis_index("c")
    base = wid * TPW
    hpw = TPW // 2
    l0 = pltpu.async_copy(xb_hbm.at[pl.ds(base, hpw)], xr0, lsem)
    l1 = pltpu.async_copy(xb_hbm.at[pl.ds(base + hpw, hpw)], xr1, lsem)
    pltpu.sync_copy(p0_hbm.at[pl.ds(base, hpw)], i0a)
    pltpu.sync_copy(p0_hbm.at[pl.ds(base + hpw, hpw)], i0b)
    pltpu.sync_copy(p1_hbm.at[pl.ds(base, hpw)], i1a)
    pltpu.sync_copy(p1_hbm.at[pl.ds(base + hpw, hpw)], i1b)
    pltpu.sync_copy(w0s_hbm.at[pl.ds(base, hpw)], wr0a)
    pltpu.sync_copy(w0s_hbm.at[pl.ds(base + hpw, hpw)], wr0b)
    pltpu.sync_copy(w1s_hbm.at[pl.ds(base, hpw)], wr1a)
    pltpu.sync_copy(w1s_hbm.at[pl.ds(base + hpw, hpw)], wr1b)
    l0.wait()
    c0 = pltpu.async_copy(xr0, xg_hbm.at[i0a], sem)
    c1 = pltpu.async_copy(xr0, xg_hbm.at[i1a], sem)
    l1.wait()
    c2 = pltpu.async_copy(xr1, xg_hbm.at[i0b], sem)
    c3 = pltpu.async_copy(xr1, xg_hbm.at[i1b], sem)
    c4 = pltpu.async_copy(wr0a, wg_hbm.at[i0a], sem)
    c5 = pltpu.async_copy(wr1a, wg_hbm.at[i1a], sem)
    c6 = pltpu.async_copy(wr0b, wg_hbm.at[i0b], sem)
    c7 = pltpu.async_copy(wr1b, wg_hbm.at[i1b], sem)
    for c in (c0, c1, c2, c3, c4, c5, c6, c7):
        c.wait()


def _run_scatter(xbi, p0, p1, w0s, w1s):
    mesh = plsc.VectorSubcoreMesh(core_axis_name="c", subcore_axis_name="s")
    f = pl.kernel(
        _scatter_kernel,
        mesh=mesh,
        out_type=[
            jax.ShapeDtypeStruct((P, H), jnp.float32),
            jax.ShapeDtypeStruct((P, WREP), jnp.float32),
        ],
        scratch_types=[
            pltpu.VMEM((TPW // 2, H), jnp.float32),
            pltpu.VMEM((TPW // 2, H), jnp.float32),
            pltpu.VMEM((TPW // 2,), jnp.int32),
            pltpu.VMEM((TPW // 2,), jnp.int32),
            pltpu.VMEM((TPW // 2,), jnp.int32),
            pltpu.VMEM((TPW // 2,), jnp.int32),
            pltpu.VMEM((TPW // 2, WREP), jnp.float32),
            pltpu.VMEM((TPW // 2, WREP), jnp.float32),
            pltpu.VMEM((TPW // 2, WREP), jnp.float32),
            pltpu.VMEM((TPW // 2, WREP), jnp.float32),
            pltpu.SemaphoreType.DMA,
            pltpu.SemaphoreType.DMA,
        ],
    )
    return f(xbi, p0, p1, w0s, w1s)


def _ffn_kernel(blk_ref, xg_ref, wg_ref, w1_ref, w3_ref, w2_ref, yg_ref):
    xb = xg_ref[...]
    h1 = lax.dot_general(xb, w1_ref[0], (((1,), (1,)), ((), ())),
                         preferred_element_type=jnp.float32)
    h3 = lax.dot_general(xb, w3_ref[0], (((1,), (1,)), ((), ())),
                         preferred_element_type=jnp.float32)
    he = (h1 * jax.nn.sigmoid(h1)) * h3
    ye = lax.dot_general(he, w2_ref[0], (((1,), (1,)), ((), ())),
                         preferred_element_type=jnp.float32)
    yg_ref[...] = ye * wg_ref[:, :1]


def _run_ffn(blk, xgb, wg, w1, w3, w2):
    grid_spec = pltpu.PrefetchScalarGridSpec(
        num_scalar_prefetch=1,
        grid=(NB,),
        in_specs=[
            pl.BlockSpec((BK, H), lambda j, blk: (j, 0)),
            pl.BlockSpec((BK, WREP), lambda j, blk: (j, 0)),
            pl.BlockSpec((1, FF, H), lambda j, blk: (blk[j], 0, 0)),
            pl.BlockSpec((1, FF, H), lambda j, blk: (blk[j], 0, 0)),
            pl.BlockSpec((1, H, FF), lambda j, blk: (blk[j], 0, 0)),
        ],
        out_specs=pl.BlockSpec((BK, H), lambda j, blk: (j, 0)),
    )
    return pl.pallas_call(
        _ffn_kernel,
        grid_spec=grid_spec,
        out_shape=jax.ShapeDtypeStruct((P, H), jnp.float32),
    )(blk, xgb, wg, w1, w3, w2)


def _combine_kernel(yg_hbm, p0_hbm, p1_hbm, out_hbm,
                    ia, ib, b0a, b0b, b1a, b1b, ob0, ob1, sem, osem):
    wid = lax.axis_index("s") * NC + lax.axis_index("c")
    base = wid * TPW
    nch = TPW // CH
    pltpu.sync_copy(p0_hbm.at[pl.ds(base, TPW)], ia)
    pltpu.sync_copy(p1_hbm.at[pl.ds(base, TPW)], ib)
    bufs = ((b0a, b0b, ob0), (b1a, b1b, ob1))
    pend = {}
    pend[0] = (pltpu.async_copy(yg_hbm.at[ia.at[pl.ds(0, CH)]], b0a, sem),
               pltpu.async_copy(yg_hbm.at[ib.at[pl.ds(0, CH)]], b0b, sem))
    st = []
    for cidx in range(nch):
        ba, bb, ob = bufs[cidx % 2]
        if cidx + 1 < nch:
            na, nb, _ = bufs[(cidx + 1) % 2]
            s = (cidx + 1) * CH
            pend[cidx + 1] = (
                pltpu.async_copy(yg_hbm.at[ia.at[pl.ds(s, CH)]], na, sem),
                pltpu.async_copy(yg_hbm.at[ib.at[pl.ds(s, CH)]], nb, sem))
        ga, gb = pend.pop(cidx)
        ga.wait()
        gb.wait()
        if len(st) >= 2:
            st.pop(0).wait()

        def body(k, _):
            s = k * LANES
            for r in range(CH):
                ob[r, pl.ds(s, LANES)] = (ba[r, pl.ds(s, LANES)]
                                          + bb[r, pl.ds(s, LANES)])
            return 0

        lax.fori_loop(0, H // LANES, body, 0)
        st.append(pltpu.async_copy(
            ob, out_hbm.at[pl.ds(base + cidx * CH, CH)], osem))
    for h in st:
        h.wait()


def _run_combine(ygi, p0, p1):
    mesh = plsc.VectorSubcoreMesh(core_axis_name="c", subcore_axis_name="s")
    f = pl.kernel(
        _combine_kernel,
        mesh=mesh,
        out_type=jax.ShapeDtypeStruct((T, H), jnp.float32),
        scratch_types=[
            pltpu.VMEM((TPW,), jnp.int32),
            pltpu.VMEM((TPW,), jnp.int32),
            pltpu.VMEM((CH, H), jnp.float32),
            pltpu.VMEM((CH, H), jnp.float32),
            pltpu.VMEM((CH, H), jnp.float32),
            pltpu.VMEM((CH, H), jnp.float32),
            pltpu.VMEM((CH, H), jnp.float32),
            pltpu.VMEM((CH, H), jnp.float32),
            pltpu.SemaphoreType.DMA,
            pltpu.SemaphoreType.DMA,
        ],
    )
    return f(ygi, p0, p1)


def kernel(hidden_states, gate_w, w1, w3, w2):
    b, s, h = hidden_states.shape
    x = hidden_states.reshape(-1, h)
    logits, p0, p1, w0s, w1s, blk = _run_router(x, gate_w)
    p0 = p0.reshape(T)
    p1 = p1.reshape(T)
    blk = blk.reshape(NB)
    xg, wg = _run_scatter(x, p0, p1, w0s, w1s)
    yg = _run_ffn(blk, xg, wg, w1, w3, w2)
    out = _run_combine(yg, p0, p1)
    return out.reshape(b, s, h), logits
